# baseline (speedup 1.0000x reference)
import functools
import jax
import jax.numpy as jnp
from jax import lax
from jax.experimental import pallas as pl
from jax.experimental.pallas import tpu as pltpu
from jax.experimental.pallas import tpu_sc as plsc

_N = 10000
_E = 320000
_FIN = 128
_H = 112
_T = 16
_TN = 10
_C = 8

_NC = 2
_NS = 16
_NW = _NC * _NS
_EPW = _E // _NW
_ECH = 100
_NECH = _EPW // _ECH
_NBUF = 2
_RCH = 16
_NRCH = _N // _RCH
_DW = 16
_W = 128

_BLK = 1000
_NBLK = _N // _BLK

_SC_PARAMS = dict(compiler_params=pltpu.CompilerParams(use_tc_tiling_on_sc=False))


@functools.lru_cache(maxsize=None)
def _get_mesh():
    return plsc.VectorSubcoreMesh(core_axis_name="c", subcore_axis_name="s",
                                  num_cores=_NC, num_subcores=_NS)


def _zero_vmem_2d(ref, rows, cols):
    def body(i, _):
        for c in range(cols // 16):
            ref[i, pl.ds(c * 16, 16)] = jnp.zeros((16,), jnp.float32)
        return 0
    lax.fori_loop(0, rows, body, 0)


def _round_robin(sid, nch, fn):
    def body(k, _):
        c = sid + k * _NS
        @pl.when(c < nch)
        def _():
            fn(c)
        return 0
    lax.fori_loop(0, (nch + _NS - 1) // _NS, body, 0)


@functools.lru_cache(maxsize=None)
def _build_sc_degree():
    return functools.partial(
        pl.kernel,
        mesh=_get_mesh(),
        out_type=jax.ShapeDtypeStruct((_NC, _N, _DW), jnp.float32),
        scratch_types=[
            pltpu.VMEM((_NECH, _ECH), jnp.int32),
            pltpu.VMEM((_ECH, _DW), jnp.float32),
            pltpu.VMEM((_RCH, _DW), jnp.float32),
            pltpu.VMEM_SHARED((_N, _DW), jnp.float32),
        ],
        **_SC_PARAMS,
    )(_sc_degree_body)


def _sc_degree(dsts):
    return _build_sc_degree()(dsts)


def _sc_degree_body(dsts, out, idx_d, ones_v, zbuf, acc):
    cid = lax.axis_index("c")
    sid = lax.axis_index("s")
    def setup(i, _):
        for c in range(_DW // 16):
            ones_v[i, pl.ds(c * 16, 16)] = jnp.ones((16,), jnp.float32)
            zbuf[i, pl.ds(c * 16, 16)] = jnp.zeros((16,), jnp.float32)
        return 0
    lax.fori_loop(0, _ECH, setup, 0)

    _round_robin(sid, _NRCH,
                 lambda c: pltpu.sync_copy(zbuf, acc.at[pl.ds(c * _RCH, _RCH)]))
    pltpu.sync_copy(dsts.at[cid, sid], idx_d)
    plsc.subcore_barrier()

    def go(j, _):
        pltpu.sync_copy(ones_v, acc.at[idx_d.at[j]], add=True)
        return 0
    lax.fori_loop(0, _NECH, go, 0)
    plsc.subcore_barrier()

    _round_robin(sid, _NRCH,
                 lambda c: pltpu.sync_copy(acc.at[pl.ds(c * _RCH, _RCH)],
                                           out.at[cid, pl.ds(c * _RCH, _RCH)]))


@functools.lru_cache(maxsize=None)
def _build_sc_segsum():
    return functools.partial(
        pl.kernel,
        mesh=_get_mesh(),
        out_type=jax.ShapeDtypeStruct((_NC, _N, _W), jnp.float32),
        scratch_types=[
            pltpu.VMEM((_NECH, _ECH), jnp.int32),
            pltpu.VMEM((_NECH, _ECH), jnp.int32),
        ] + [pltpu.VMEM((_ECH, _W), jnp.float32) for _ in range(_NBUF)] + [
            pltpu.VMEM((_RCH, _W), jnp.float32),
            pltpu.VMEM_SHARED((_N, _W), jnp.float32),
        ] + [pltpu.SemaphoreType.DMA for _ in range(_NBUF)],
        **_SC_PARAMS,
    )(_sc_segsum_body)


def _sc_segsum(table, srcs, dsts):
    return _build_sc_segsum()(table, srcs, dsts)


def _sc_segsum_body(table, srcs, dsts, out, idx_s, idx_d, *rest):
    rows = rest[:_NBUF]
    zbuf, acc = rest[_NBUF], rest[_NBUF + 1]
    sems = rest[_NBUF + 2:]
    cid = lax.axis_index("c")
    sid = lax.axis_index("s")
    _zero_vmem_2d(zbuf, _RCH, _W)
    _round_robin(sid, _NRCH,
                 lambda c: pltpu.sync_copy(zbuf, acc.at[pl.ds(c * _RCH, _RCH)]))
    pltpu.sync_copy(srcs.at[cid, sid], idx_s)
    pltpu.sync_copy(dsts.at[cid, sid], idx_d)
    plsc.subcore_barrier()

    for b in range(_NBUF):
        pltpu.async_copy(table.at[idx_s.at[b]], rows[b], sems[b])

    def go(jo, _):
        for b in range(_NBUF):
            jj = jo * _NBUF + b
            pltpu.make_async_copy(table.at[idx_s.at[jj]], rows[b], sems[b]).wait()
            pltpu.sync_copy(rows[b], acc.at[idx_d.at[jj]], add=True)
            nxt = jj + _NBUF
            @pl.when(nxt < _NECH)
            def _():
                pltpu.async_copy(table.at[idx_s.at[nxt]], rows[b], sems[b])
        return 0
    lax.fori_loop(0, _NECH // _NBUF, go, 0)
    plsc.subcore_barrier()

    _round_robin(sid, _NRCH,
                 lambda c: pltpu.sync_copy(acc.at[pl.ds(c * _RCH, _RCH)],
                                           out.at[cid, pl.ds(c * _RCH, _RCH)]))


_DOT = dict(preferred_element_type=jnp.float32, precision=lax.Precision.HIGHEST)


def _tc_pre_body(x_ref, w1_ref, degp_ref, u_ref, dinv_ref, indeg_ref, sdeg_ref):
    pid = pl.program_id(0)
    indeg = degp_ref[0, :, 0:1] + degp_ref[1, :, 0:1]
    dinv = lax.rsqrt(indeg + 1.0)
    h = jnp.dot(x_ref[...], w1_ref[...], **_DOT)
    u_ref[...] = jnp.concatenate(
        [h * dinv, jnp.zeros((_BLK, _W - _H), jnp.float32)], axis=1)
    dinv_ref[...] = dinv
    indeg_ref[...] = indeg
    @pl.when(pid == 0)
    def _():
        sdeg_ref[...] = jnp.zeros_like(sdeg_ref)
    sdeg_ref[...] += jnp.sum(jnp.maximum(indeg, 1.0)).reshape(1, 1)


def _tc_relu_body(s1_ref, u_ref, dinv_ref, b1_ref, h_ref):
    s = (s1_ref[0] + s1_ref[1] + u_ref[...])[:, :_H]
    h = jnp.maximum(dinv_ref[...] * s + b1_ref[...], 0.0)
    h_ref[...] = jnp.concatenate(
        [h, jnp.zeros((_BLK, _W - _H), jnp.float32)], axis=1)


def _tc_de_body(h_ref, s2_ref, indeg_ref, sdeg_ref, tfeat_ref, tadj_ref,
                alpha0_ref, gamma_ref, beta_ref, w2_ref, dinv_ref,
                u2_ref, ysc, stat_sc):
    ph = pl.program_id(0)
    pid = pl.program_id(1)

    @pl.when(ph == 0)
    def _():
        indeg = indeg_ref[...]
        degc = jnp.maximum(indeg, 1.0)
        nb = (s2_ref[0] + s2_ref[1])[:, :_H] / degc
        tfeat = tfeat_ref[...]
        tmean = jnp.mean(tfeat, axis=1)
        msq = jnp.mean(jnp.sum(tfeat * tfeat, axis=2), axis=1)
        cross = lax.dot_general(nb, tmean, (((1,), (1,)), ((), ())), **_DOT)
        feat = (jnp.sum(nb * nb, axis=1, keepdims=True)
                - 2.0 * cross + msq[None, :])
        tstruct = jnp.mean(tadj_ref[...], axis=(1, 2))
        deg_norm = indeg * (_N / sdeg_ref[0, 0])
        struct = (deg_norm - tstruct[None, :]) ** 2
        alpha = jax.nn.sigmoid(alpha0_ref[0, 0])
        y = jnp.exp(-(alpha * feat + (1.0 - alpha) * struct))
        ysc[pl.ds(pid * _BLK, _BLK), :] = y
        h = h_ref[:, :_H]
        row0 = jnp.concatenate([jnp.sum(h, axis=0), jnp.sum(y, axis=0)])
        row1 = jnp.concatenate([jnp.sum(h * h, axis=0), jnp.sum(y * y, axis=0)])
        @pl.when(pid == 0)
        def _():
            stat_sc[...] = jnp.zeros_like(stat_sc)
        stat_sc[...] += jnp.stack([row0, row1])

    @pl.when(ph == 1)
    def _():
        mean = stat_sc[0:1, :] / _N
        var = stat_sc[1:2, :] / _N - mean * mean
        scale = lax.rsqrt(var + 1e-5) * gamma_ref[...]
        shift = beta_ref[...] - mean * scale
        z = jnp.concatenate([h_ref[:, :_H], ysc[pl.ds(pid * _BLK, _BLK), :]],
                            axis=1)
        zn = z * scale + shift
        p = jnp.dot(zn, w2_ref[...], **_DOT)
        u2_ref[...] = jnp.concatenate(
            [p * dinv_ref[...], jnp.zeros((_BLK, _W - _H), jnp.float32)],
            axis=1)


def _tc_final_body(s3_ref, u2_ref, dinv_ref, b2_ref, wl_ref, bl_ref,
                   out_ref, h2_ref):
    s = (s3_ref[0] + s3_ref[1] + u2_ref[...])[:, :_H]
    h2 = jnp.maximum(dinv_ref[...] * s + b2_ref[...], 0.0)
    h2_ref[...] = h2
    out_ref[...] = jnp.dot(h2, wl_ref[...], **_DOT) + bl_ref[...]


def _row_spec(cols):
    return pl.BlockSpec((_BLK, cols), lambda i: (i, 0))


def _part_spec(cols):
    return pl.BlockSpec((_NC, _BLK, cols), lambda i: (0, i, 0))


def _full_spec(shape):
    rank = len(shape)
    return pl.BlockSpec(shape, lambda i, _r=rank: (0,) * _r)


def kernel(x, edge_index, W1, b1, tfeat, tadj, alpha0, gamma, beta, W2, b2,
           Wlin, blin):
    f32 = jnp.float32
    src = edge_index[0].reshape(_NC, _NS, _NECH, _ECH).astype(jnp.int32)
    dst = edge_index[1].reshape(_NC, _NS, _NECH, _ECH).astype(jnp.int32)
    b1r = b1.reshape(1, _H)
    b2r = b2.reshape(1, _H)
    blr = blin.reshape(1, _C)
    gr = gamma.reshape(1, _H + _T)
    br = beta.reshape(1, _H + _T)
    a0 = alpha0.reshape(1, 1)

    degp = _sc_degree(dst)

    u, dinv, indeg, sdeg = pl.pallas_call(
        _tc_pre_body,
        grid=(_NBLK,),
        in_specs=[_row_spec(_FIN), _full_spec(W1.shape), _part_spec(_DW)],
        out_specs=[_row_spec(_W), _row_spec(1), _row_spec(1),
                   pl.BlockSpec((1, 1), lambda i: (0, 0))],
        out_shape=[jax.ShapeDtypeStruct((_N, _W), f32),
                   jax.ShapeDtypeStruct((_N, 1), f32),
                   jax.ShapeDtypeStruct((_N, 1), f32),
                   jax.ShapeDtypeStruct((1, 1), f32)],
    )(x, W1, degp)

    s1 = _sc_segsum(u, src, dst)

    h = pl.pallas_call(
        _tc_relu_body,
        grid=(_NBLK,),
        in_specs=[_part_spec(_W), _row_spec(_W), _row_spec(1),
                  _full_spec((1, _H))],
        out_specs=_row_spec(_W),
        out_shape=jax.ShapeDtypeStruct((_N, _W), f32),
    )(s1, u, dinv, b1r)

    s2 = _sc_segsum(h, src, dst)

    def _r2(cols):
        return pl.BlockSpec((_BLK, cols), lambda p, i: (i, 0))

    def _p2(cols):
        return pl.BlockSpec((_NC, _BLK, cols), lambda p, i: (0, i, 0))

    def _f2(shape):
        rank = len(shape)
        return pl.BlockSpec(shape, lambda p, i, _r=rank: (0,) * _r)

    u2 = pl.pallas_call(
        _tc_de_body,
        grid=(2, _NBLK),
        in_specs=[_r2(_W), _p2(_W), _r2(1), _f2((1, 1)), _f2(tfeat.shape),
                  _f2(tadj.shape), _f2((1, 1)), _f2((1, _H + _T)),
                  _f2((1, _H + _T)), _f2(W2.shape), _r2(1)],
        out_specs=_r2(_W),
        out_shape=jax.ShapeDtypeStruct((_N, _W), f32),
        scratch_shapes=[pltpu.VMEM((_N, _T), f32),
                        pltpu.VMEM((2, _H + _T), f32)],
    )(h, s2, indeg, sdeg, tfeat, tadj, a0, gr, br, W2, dinv)

    s3 = _sc_segsum(u2, src, dst)

    out, h2 = pl.pallas_call(
        _tc_final_body,
        grid=(_NBLK,),
        in_specs=[_part_spec(_W), _row_spec(_W), _row_spec(1),
                  _full_spec((1, _H)), _full_spec(Wlin.shape),
                  _full_spec((1, _C))],
        out_specs=[_row_spec(_C), _row_spec(_H)],
        out_shape=[jax.ShapeDtypeStruct((_N, _C), f32),
                   jax.ShapeDtypeStruct((_N, _H), f32)],
    )(s3, u2, dinv, b2r, Wlin, blr)

    return (out, h2)


# speedup vs baseline: 1.0516x; 1.0516x over previous
import functools
import jax
import jax.numpy as jnp
from jax import lax
from jax.experimental import pallas as pl
from jax.experimental.pallas import tpu as pltpu
from jax.experimental.pallas import tpu_sc as plsc

_N = 10000
_E = 320000
_FIN = 128
_H = 112
_T = 16
_TN = 10
_C = 8

_NC = 2
_NS = 16
_NW = _NC * _NS
_EPW = _E // _NW
_ECH = 100
_NECH = _EPW // _ECH
_NBUF = 2
_RCH = 40
_NRCH = _N // _RCH
_DW = 16
_W = _H

_BLK = 1000
_NBLK = _N // _BLK

_SC_PARAMS = dict(compiler_params=pltpu.CompilerParams(use_tc_tiling_on_sc=False))


@functools.lru_cache(maxsize=None)
def _get_mesh():
    return plsc.VectorSubcoreMesh(core_axis_name="c", subcore_axis_name="s",
                                  num_cores=_NC, num_subcores=_NS)


def _zero_vmem_2d(ref, rows, cols):
    def body(i, _):
        for c in range(cols // 16):
            ref[i, pl.ds(c * 16, 16)] = jnp.zeros((16,), jnp.float32)
        return 0
    lax.fori_loop(0, rows, body, 0)


def _round_robin(sid, nch, fn):
    def body(k, _):
        c = sid + k * _NS
        @pl.when(c < nch)
        def _():
            fn(c)
        return 0
    lax.fori_loop(0, (nch + _NS - 1) // _NS, body, 0)


@functools.lru_cache(maxsize=None)
def _build_sc_degree():
    return functools.partial(
        pl.kernel,
        mesh=_get_mesh(),
        out_type=jax.ShapeDtypeStruct((_NC, _N, _DW), jnp.float32),
        scratch_types=[
            pltpu.VMEM((_NECH, _ECH), jnp.int32),
            pltpu.VMEM((_ECH, _DW), jnp.float32),
            pltpu.VMEM((_RCH, _DW), jnp.float32),
            pltpu.VMEM_SHARED((_N, _DW), jnp.float32),
        ],
        **_SC_PARAMS,
    )(_sc_degree_body)


def _sc_degree(dsts):
    return _build_sc_degree()(dsts)


def _sc_degree_body(dsts, out, idx_d, ones_v, zbuf, acc):
    cid = lax.axis_index("c")
    sid = lax.axis_index("s")
    def setup(i, _):
        for c in range(_DW // 16):
            ones_v[i, pl.ds(c * 16, 16)] = jnp.ones((16,), jnp.float32)
            zbuf[i, pl.ds(c * 16, 16)] = jnp.zeros((16,), jnp.float32)
        return 0
    lax.fori_loop(0, _ECH, setup, 0)

    _round_robin(sid, _NRCH,
                 lambda c: pltpu.sync_copy(zbuf, acc.at[pl.ds(c * _RCH, _RCH)]))
    pltpu.sync_copy(dsts.at[cid, sid], idx_d)
    plsc.subcore_barrier()

    def go(j, _):
        pltpu.sync_copy(ones_v, acc.at[idx_d.at[j]], add=True)
        return 0
    lax.fori_loop(0, _NECH, go, 0)
    plsc.subcore_barrier()

    _round_robin(sid, _NRCH,
                 lambda c: pltpu.sync_copy(acc.at[pl.ds(c * _RCH, _RCH)],
                                           out.at[cid, pl.ds(c * _RCH, _RCH)]))


@functools.lru_cache(maxsize=None)
def _build_sc_segsum():
    return functools.partial(
        pl.kernel,
        mesh=_get_mesh(),
        out_type=jax.ShapeDtypeStruct((_NC, _N, _W), jnp.float32),
        scratch_types=[
            pltpu.VMEM((_NECH, _ECH), jnp.int32),
            pltpu.VMEM((_NECH, _ECH), jnp.int32),
        ] + [pltpu.VMEM((_ECH, _W), jnp.float32) for _ in range(_NBUF)] + [
            pltpu.VMEM((_RCH, _W), jnp.float32),
            pltpu.VMEM_SHARED((_N, _W), jnp.float32),
        ] + [pltpu.SemaphoreType.DMA for _ in range(_NBUF)],
        **_SC_PARAMS,
    )(_sc_segsum_body)


def _sc_segsum(table, srcs, dsts):
    return _build_sc_segsum()(table, srcs, dsts)


def _sc_segsum_body(table, srcs, dsts, out, idx_s, idx_d, *rest):
    rows = rest[:_NBUF]
    zbuf, acc = rest[_NBUF], rest[_NBUF + 1]
    sems = rest[_NBUF + 2:]
    cid = lax.axis_index("c")
    sid = lax.axis_index("s")
    _zero_vmem_2d(zbuf, _RCH, _W)
    _round_robin(sid, _NRCH,
                 lambda c: pltpu.sync_copy(zbuf, acc.at[pl.ds(c * _RCH, _RCH)]))
    pltpu.sync_copy(srcs.at[cid, sid], idx_s)
    pltpu.sync_copy(dsts.at[cid, sid], idx_d)
    plsc.subcore_barrier()

    for b in range(_NBUF):
        pltpu.async_copy(table.at[idx_s.at[b]], rows[b], sems[b])

    def go(jo, _):
        for b in range(_NBUF):
            jj = jo * _NBUF + b
            pltpu.make_async_copy(table.at[idx_s.at[jj]], rows[b], sems[b]).wait()
            pltpu.sync_copy(rows[b], acc.at[idx_d.at[jj]], add=True)
            nxt = jj + _NBUF
            @pl.when(nxt < _NECH)
            def _():
                pltpu.async_copy(table.at[idx_s.at[nxt]], rows[b], sems[b])
        return 0
    lax.fori_loop(0, _NECH // _NBUF, go, 0)
    plsc.subcore_barrier()

    _round_robin(sid, _NRCH,
                 lambda c: pltpu.sync_copy(acc.at[pl.ds(c * _RCH, _RCH)],
                                           out.at[cid, pl.ds(c * _RCH, _RCH)]))


_DOT = dict(preferred_element_type=jnp.float32, precision=lax.Precision.HIGHEST)


def _tc_pre_body(x_ref, w1_ref, degp_ref, u_ref, dinv_ref, indeg_ref,
                 sdeg_ref):
    pid = pl.program_id(0)
    indeg = degp_ref[0, :, 0:1] + degp_ref[1, :, 0:1]
    dinv = lax.rsqrt(indeg + 1.0)
    u_ref[...] = jnp.dot(x_ref[...], w1_ref[...], **_DOT) * dinv
    dinv_ref[...] = dinv
    indeg_ref[...] = indeg
    @pl.when(pid == 0)
    def _():
        sdeg_ref[...] = jnp.zeros_like(sdeg_ref)
    sdeg_ref[...] += jnp.sum(jnp.maximum(indeg, 1.0)).reshape(1, 1)


def _tc_relu_body(s1_ref, u_ref, dinv_ref, b1_ref, h_ref):
    s = s1_ref[0] + s1_ref[1] + u_ref[...]
    h_ref[...] = jnp.maximum(dinv_ref[...] * s + b1_ref[...], 0.0)


def _tc_de_body(h_ref, s2_ref, indeg_ref, sdeg_ref, tfeat_ref, tadj_ref,
                alpha0_ref, gamma_ref, beta_ref, w2_ref, dinv_ref,
                u2_ref, ysc, stat_sc):
    ph = pl.program_id(0)
    pid = pl.program_id(1)

    @pl.when(ph == 0)
    def _():
        indeg = indeg_ref[...]
        degc = jnp.maximum(indeg, 1.0)
        nb = (s2_ref[0] + s2_ref[1]) / degc
        tfeat = tfeat_ref[...]
        tmean = jnp.mean(tfeat, axis=1)
        msq = jnp.mean(jnp.sum(tfeat * tfeat, axis=2), axis=1)
        cross = lax.dot_general(nb, tmean, (((1,), (1,)), ((), ())), **_DOT)
        feat = (jnp.sum(nb * nb, axis=1, keepdims=True)
                - 2.0 * cross + msq[None, :])
        tstruct = jnp.mean(tadj_ref[...], axis=(1, 2))
        deg_norm = indeg * (_N / sdeg_ref[0, 0])
        struct = (deg_norm - tstruct[None, :]) ** 2
        alpha = jax.nn.sigmoid(alpha0_ref[0, 0])
        y = jnp.exp(-(alpha * feat + (1.0 - alpha) * struct))
        ysc[pl.ds(pid * _BLK, _BLK), :] = y
        h = h_ref[...]
        row0 = jnp.concatenate([jnp.sum(h, axis=0), jnp.sum(y, axis=0)])
        row1 = jnp.concatenate([jnp.sum(h * h, axis=0), jnp.sum(y * y, axis=0)])
        @pl.when(pid == 0)
        def _():
            stat_sc[...] = jnp.zeros_like(stat_sc)
        stat_sc[...] += jnp.stack([row0, row1])

    @pl.when(ph == 1)
    def _():
        mean = stat_sc[0:1, :] / _N
        var = stat_sc[1:2, :] / _N - mean * mean
        scale = lax.rsqrt(var + 1e-5) * gamma_ref[...]
        shift = beta_ref[...] - mean * scale
        z = jnp.concatenate([h_ref[...], ysc[pl.ds(pid * _BLK, _BLK), :]],
                            axis=1)
        zn = z * scale + shift
        p = jnp.dot(zn, w2_ref[...], **_DOT)
        u2_ref[...] = p * dinv_ref[...]


def _tc_final_body(s3_ref, u2_ref, dinv_ref, b2_ref, wl_ref, bl_ref,
                   out_ref, h2_ref):
    s = s3_ref[0] + s3_ref[1] + u2_ref[...]
    h2 = jnp.maximum(dinv_ref[...] * s + b2_ref[...], 0.0)
    h2_ref[...] = h2
    out_ref[...] = jnp.dot(h2, wl_ref[...], **_DOT) + bl_ref[...]


def _row_spec(cols):
    return pl.BlockSpec((_BLK, cols), lambda i: (i, 0))


def _part_spec(cols):
    return pl.BlockSpec((_NC, _BLK, cols), lambda i: (0, i, 0))


def _full_spec(shape):
    rank = len(shape)
    return pl.BlockSpec(shape, lambda i, _r=rank: (0,) * _r)


def kernel(x, edge_index, W1, b1, tfeat, tadj, alpha0, gamma, beta, W2, b2,
           Wlin, blin):
    f32 = jnp.float32
    src = edge_index[0].reshape(_NC, _NS, _NECH, _ECH).astype(jnp.int32)
    dst = edge_index[1].reshape(_NC, _NS, _NECH, _ECH).astype(jnp.int32)
    b1r = b1.reshape(1, _H)
    b2r = b2.reshape(1, _H)
    blr = blin.reshape(1, _C)
    gr = gamma.reshape(1, _H + _T)
    br = beta.reshape(1, _H + _T)
    a0 = alpha0.reshape(1, 1)

    degp = _sc_degree(dst)

    u, dinv, indeg, sdeg = pl.pallas_call(
        _tc_pre_body,
        grid=(_NBLK,),
        in_specs=[_row_spec(_FIN), _full_spec(W1.shape), _part_spec(_DW)],
        out_specs=[_row_spec(_H), _row_spec(1), _row_spec(1),
                   pl.BlockSpec((1, 1), lambda i: (0, 0))],
        out_shape=[jax.ShapeDtypeStruct((_N, _H), f32),
                   jax.ShapeDtypeStruct((_N, 1), f32),
                   jax.ShapeDtypeStruct((_N, 1), f32),
                   jax.ShapeDtypeStruct((1, 1), f32)],
    )(x, W1, degp)

    s1 = _sc_segsum(u, src, dst)

    h = pl.pallas_call(
        _tc_relu_body,
        grid=(_NBLK,),
        in_specs=[_part_spec(_H), _row_spec(_H), _row_spec(1),
                  _full_spec((1, _H))],
        out_specs=_row_spec(_H),
        out_shape=jax.ShapeDtypeStruct((_N, _H), f32),
    )(s1, u, dinv, b1r)

    s2 = _sc_segsum(h, src, dst)

    def _r2(cols):
        return pl.BlockSpec((_BLK, cols), lambda p, i: (i, 0))

    def _p2(cols):
        return pl.BlockSpec((_NC, _BLK, cols), lambda p, i: (0, i, 0))

    def _f2(shape):
        rank = len(shape)
        return pl.BlockSpec(shape, lambda p, i, _r=rank: (0,) * _r)

    u2 = pl.pallas_call(
        _tc_de_body,
        grid=(2, _NBLK),
        in_specs=[_r2(_H), _p2(_H), _r2(1), _f2((1, 1)), _f2(tfeat.shape),
                  _f2(tadj.shape), _f2((1, 1)), _f2((1, _H + _T)),
                  _f2((1, _H + _T)), _f2(W2.shape), _r2(1)],
        out_specs=_r2(_H),
        out_shape=jax.ShapeDtypeStruct((_N, _H), f32),
        scratch_shapes=[pltpu.VMEM((_N, _T), f32),
                        pltpu.VMEM((2, _H + _T), f32)],
    )(h, s2, indeg, sdeg, tfeat, tadj, a0, gr, br, W2, dinv)

    s3 = _sc_segsum(u2, src, dst)

    out, h2 = pl.pallas_call(
        _tc_final_body,
        grid=(_NBLK,),
        in_specs=[_part_spec(_H), _row_spec(_H), _row_spec(1),
                  _full_spec((1, _H)), _full_spec(Wlin.shape),
                  _full_spec((1, _C))],
        out_specs=[_row_spec(_C), _row_spec(_H)],
        out_shape=[jax.ShapeDtypeStruct((_N, _C), f32),
                   jax.ShapeDtypeStruct((_N, _H), f32)],
    )(s3, u2, dinv, b2r, Wlin, blr)

    return (out, h2)


# speedup vs baseline: 1.0900x; 1.0364x over previous
import functools
import jax
import jax.numpy as jnp
from jax import lax
from jax.experimental import pallas as pl
from jax.experimental.pallas import tpu as pltpu
from jax.experimental.pallas import tpu_sc as plsc

_N = 10000
_E = 320000
_FIN = 128
_H = 112
_T = 16
_TN = 10
_C = 8

_NC = 2
_NS = 16
_NW = _NC * _NS
_EPW = _E // _NW
_ECH = 125
_NECH = _EPW // _ECH
_NBUF = 2
_RCH = 40
_NRCH = _N // _RCH
_DW = 16
_W = _H

_BLK = 1000
_NBLK = _N // _BLK

_SC_PARAMS = dict(compiler_params=pltpu.CompilerParams(use_tc_tiling_on_sc=False))


@functools.lru_cache(maxsize=None)
def _get_mesh():
    return plsc.VectorSubcoreMesh(core_axis_name="c", subcore_axis_name="s",
                                  num_cores=_NC, num_subcores=_NS)


def _zero_vmem_2d(ref, rows, cols):
    def body(i, _):
        for c in range(cols // 16):
            ref[i, pl.ds(c * 16, 16)] = jnp.zeros((16,), jnp.float32)
        return 0
    lax.fori_loop(0, rows, body, 0)


def _round_robin(sid, nch, fn):
    def body(k, _):
        c = sid + k * _NS
        @pl.when(c < nch)
        def _():
            fn(c)
        return 0
    lax.fori_loop(0, (nch + _NS - 1) // _NS, body, 0)


@functools.lru_cache(maxsize=None)
def _build_sc_degree():
    return functools.partial(
        pl.kernel,
        mesh=_get_mesh(),
        out_type=jax.ShapeDtypeStruct((_NC, _N, _DW), jnp.float32),
        scratch_types=[
            pltpu.VMEM((_NECH, _ECH), jnp.int32),
            pltpu.VMEM((_ECH, _DW), jnp.float32),
            pltpu.VMEM((_RCH, _DW), jnp.float32),
            pltpu.VMEM_SHARED((_N, _DW), jnp.float32),
        ],
        **_SC_PARAMS,
    )(_sc_degree_body)


def _sc_degree(dsts):
    return _build_sc_degree()(dsts)


def _sc_degree_body(dsts, out, idx_d, ones_v, zbuf, acc):
    cid = lax.axis_index("c")
    sid = lax.axis_index("s")
    def setup(i, _):
        for c in range(_DW // 16):
            ones_v[i, pl.ds(c * 16, 16)] = jnp.ones((16,), jnp.float32)
            zbuf[i, pl.ds(c * 16, 16)] = jnp.zeros((16,), jnp.float32)
        return 0
    lax.fori_loop(0, _ECH, setup, 0)

    _round_robin(sid, _NRCH,
                 lambda c: pltpu.sync_copy(zbuf, acc.at[pl.ds(c * _RCH, _RCH)]))
    pltpu.sync_copy(dsts.at[cid, sid], idx_d)
    plsc.subcore_barrier()

    def go(j, _):
        pltpu.sync_copy(ones_v, acc.at[idx_d.at[j]], add=True)
        return 0
    lax.fori_loop(0, _NECH, go, 0)
    plsc.subcore_barrier()

    _round_robin(sid, _NRCH,
                 lambda c: pltpu.sync_copy(acc.at[pl.ds(c * _RCH, _RCH)],
                                           out.at[cid, pl.ds(c * _RCH, _RCH)]))


@functools.lru_cache(maxsize=None)
def _build_sc_segsum():
    return functools.partial(
        pl.kernel,
        mesh=_get_mesh(),
        out_type=jax.ShapeDtypeStruct((_NC, _N, _W), jnp.float32),
        scratch_types=[
            pltpu.VMEM((_NECH, _ECH), jnp.int32),
            pltpu.VMEM((_NECH, _ECH), jnp.int32),
        ] + [pltpu.VMEM((_ECH, _W), jnp.float32) for _ in range(_NBUF)] + [
            pltpu.VMEM((_RCH, _W), jnp.float32),
            pltpu.VMEM_SHARED((_N, _W), jnp.float32),
        ] + [pltpu.SemaphoreType.DMA for _ in range(_NBUF)],
        **_SC_PARAMS,
    )(_sc_segsum_body)


def _sc_segsum(table, srcs, dsts):
    return _build_sc_segsum()(table, srcs, dsts)


def _sc_segsum_body(table, srcs, dsts, out, idx_s, idx_d, *rest):
    rows = rest[:_NBUF]
    zbuf, acc = rest[_NBUF], rest[_NBUF + 1]
    sems = rest[_NBUF + 2:]
    cid = lax.axis_index("c")
    sid = lax.axis_index("s")
    _zero_vmem_2d(zbuf, _RCH, _W)
    _round_robin(sid, _NRCH,
                 lambda c: pltpu.sync_copy(zbuf, acc.at[pl.ds(c * _RCH, _RCH)]))
    pltpu.sync_copy(srcs.at[cid, sid], idx_s)
    pltpu.sync_copy(dsts.at[cid, sid], idx_d)
    plsc.subcore_barrier()

    for b in range(_NBUF):
        pltpu.async_copy(table.at[idx_s.at[b]], rows[b], sems[b])

    def go(jo, _):
        for b in range(_NBUF):
            jj = jo * _NBUF + b
            pltpu.make_async_copy(table.at[idx_s.at[jj]], rows[b], sems[b]).wait()
            pltpu.sync_copy(rows[b], acc.at[idx_d.at[jj]], add=True)
            nxt = jj + _NBUF
            @pl.when(nxt < _NECH)
            def _():
                pltpu.async_copy(table.at[idx_s.at[nxt]], rows[b], sems[b])
        return 0
    lax.fori_loop(0, _NECH // _NBUF, go, 0)
    plsc.subcore_barrier()

    _round_robin(sid, _NRCH,
                 lambda c: pltpu.sync_copy(acc.at[pl.ds(c * _RCH, _RCH)],
                                           out.at[cid, pl.ds(c * _RCH, _RCH)]))


_DOT = dict(preferred_element_type=jnp.float32, precision=lax.Precision.HIGHEST)


def _tc_pre_body(x_ref, w1_ref, degp_ref, u_ref, dinv_ref, indeg_ref,
                 sdeg_ref):
    pid = pl.program_id(0)
    indeg = degp_ref[0, :, 0:1] + degp_ref[1, :, 0:1]
    dinv = lax.rsqrt(indeg + 1.0)
    u_ref[...] = jnp.dot(x_ref[...], w1_ref[...], **_DOT) * dinv
    dinv_ref[...] = dinv
    indeg_ref[...] = indeg
    @pl.when(pid == 0)
    def _():
        sdeg_ref[...] = jnp.zeros_like(sdeg_ref)
    sdeg_ref[...] += jnp.sum(jnp.maximum(indeg, 1.0)).reshape(1, 1)


def _tc_relu_body(s1_ref, u_ref, dinv_ref, b1_ref, h_ref):
    s = s1_ref[0] + s1_ref[1] + u_ref[...]
    h_ref[...] = jnp.maximum(dinv_ref[...] * s + b1_ref[...], 0.0)


def _tc_de_body(h_ref, s2_ref, indeg_ref, sdeg_ref, tfeat_ref, tadj_ref,
                alpha0_ref, gamma_ref, beta_ref, w2_ref, dinv_ref,
                u2_ref, ysc, stat_sc):
    ph = pl.program_id(0)
    pid = pl.program_id(1)

    @pl.when(ph == 0)
    def _():
        indeg = indeg_ref[...]
        degc = jnp.maximum(indeg, 1.0)
        nb = (s2_ref[0] + s2_ref[1]) / degc
        tfeat = tfeat_ref[...]
        tmean = jnp.mean(tfeat, axis=1)
        msq = jnp.mean(jnp.sum(tfeat * tfeat, axis=2), axis=1)
        cross = lax.dot_general(nb, tmean, (((1,), (1,)), ((), ())), **_DOT)
        feat = (jnp.sum(nb * nb, axis=1, keepdims=True)
                - 2.0 * cross + msq[None, :])
        tstruct = jnp.mean(tadj_ref[...], axis=(1, 2))
        deg_norm = indeg * (_N / sdeg_ref[0, 0])
        struct = (deg_norm - tstruct[None, :]) ** 2
        alpha = jax.nn.sigmoid(alpha0_ref[0, 0])
        y = jnp.exp(-(alpha * feat + (1.0 - alpha) * struct))
        ysc[pl.ds(pid * _BLK, _BLK), :] = y
        h = h_ref[...]
        row0 = jnp.concatenate([jnp.sum(h, axis=0), jnp.sum(y, axis=0)])
        row1 = jnp.concatenate([jnp.sum(h * h, axis=0), jnp.sum(y * y, axis=0)])
        @pl.when(pid == 0)
        def _():
            stat_sc[...] = jnp.zeros_like(stat_sc)
        stat_sc[...] += jnp.stack([row0, row1])

    @pl.when(ph == 1)
    def _():
        mean = stat_sc[0:1, :] / _N
        var = stat_sc[1:2, :] / _N - mean * mean
        scale = lax.rsqrt(var + 1e-5) * gamma_ref[...]
        shift = beta_ref[...] - mean * scale
        z = jnp.concatenate([h_ref[...], ysc[pl.ds(pid * _BLK, _BLK), :]],
                            axis=1)
        zn = z * scale + shift
        p = jnp.dot(zn, w2_ref[...], **_DOT)
        u2_ref[...] = p * dinv_ref[...]


def _tc_final_body(s3_ref, u2_ref, dinv_ref, b2_ref, wl_ref, bl_ref,
                   out_ref, h2_ref):
    s = s3_ref[0] + s3_ref[1] + u2_ref[...]
    h2 = jnp.maximum(dinv_ref[...] * s + b2_ref[...], 0.0)
    h2_ref[...] = h2
    out_ref[...] = jnp.dot(h2, wl_ref[...], **_DOT) + bl_ref[...]


def _row_spec(cols):
    return pl.BlockSpec((_BLK, cols), lambda i: (i, 0))


def _part_spec(cols):
    return pl.BlockSpec((_NC, _BLK, cols), lambda i: (0, i, 0))


def _full_spec(shape):
    rank = len(shape)
    return pl.BlockSpec(shape, lambda i, _r=rank: (0,) * _r)


def kernel(x, edge_index, W1, b1, tfeat, tadj, alpha0, gamma, beta, W2, b2,
           Wlin, blin):
    f32 = jnp.float32
    src = edge_index[0].reshape(_NC, _NS, _NECH, _ECH).astype(jnp.int32)
    dst = edge_index[1].reshape(_NC, _NS, _NECH, _ECH).astype(jnp.int32)
    b1r = b1.reshape(1, _H)
    b2r = b2.reshape(1, _H)
    blr = blin.reshape(1, _C)
    gr = gamma.reshape(1, _H + _T)
    br = beta.reshape(1, _H + _T)
    a0 = alpha0.reshape(1, 1)

    degp = _sc_degree(dst)

    u, dinv, indeg, sdeg = pl.pallas_call(
        _tc_pre_body,
        grid=(_NBLK,),
        in_specs=[_row_spec(_FIN), _full_spec(W1.shape), _part_spec(_DW)],
        out_specs=[_row_spec(_H), _row_spec(1), _row_spec(1),
                   pl.BlockSpec((1, 1), lambda i: (0, 0))],
        out_shape=[jax.ShapeDtypeStruct((_N, _H), f32),
                   jax.ShapeDtypeStruct((_N, 1), f32),
                   jax.ShapeDtypeStruct((_N, 1), f32),
                   jax.ShapeDtypeStruct((1, 1), f32)],
    )(x, W1, degp)

    s1 = _sc_segsum(u, src, dst)

    h = pl.pallas_call(
        _tc_relu_body,
        grid=(_NBLK,),
        in_specs=[_part_spec(_H), _row_spec(_H), _row_spec(1),
                  _full_spec((1, _H))],
        out_specs=_row_spec(_H),
        out_shape=jax.ShapeDtypeStruct((_N, _H), f32),
    )(s1, u, dinv, b1r)

    s2 = _sc_segsum(h, src, dst)

    def _r2(cols):
        return pl.BlockSpec((_BLK, cols), lambda p, i: (i, 0))

    def _p2(cols):
        return pl.BlockSpec((_NC, _BLK, cols), lambda p, i: (0, i, 0))

    def _f2(shape):
        rank = len(shape)
        return pl.BlockSpec(shape, lambda p, i, _r=rank: (0,) * _r)

    u2 = pl.pallas_call(
        _tc_de_body,
        grid=(2, _NBLK),
        in_specs=[_r2(_H), _p2(_H), _r2(1), _f2((1, 1)), _f2(tfeat.shape),
                  _f2(tadj.shape), _f2((1, 1)), _f2((1, _H + _T)),
                  _f2((1, _H + _T)), _f2(W2.shape), _r2(1)],
        out_specs=_r2(_H),
        out_shape=jax.ShapeDtypeStruct((_N, _H), f32),
        scratch_shapes=[pltpu.VMEM((_N, _T), f32),
                        pltpu.VMEM((2, _H + _T), f32)],
    )(h, s2, indeg, sdeg, tfeat, tadj, a0, gr, br, W2, dinv)

    s3 = _sc_segsum(u2, src, dst)

    out, h2 = pl.pallas_call(
        _tc_final_body,
        grid=(_NBLK,),
        in_specs=[_part_spec(_H), _row_spec(_H), _row_spec(1),
                  _full_spec((1, _H)), _full_spec(Wlin.shape),
                  _full_spec((1, _C))],
        out_specs=[_row_spec(_C), _row_spec(_H)],
        out_shape=[jax.ShapeDtypeStruct((_N, _C), f32),
                   jax.ShapeDtypeStruct((_N, _H), f32)],
    )(s3, u2, dinv, b2r, Wlin, blr)

    return (out, h2)


# speedup vs baseline: 1.1516x; 1.0565x over previous
import functools
import jax
import jax.numpy as jnp
from jax import lax
from jax.experimental import pallas as pl
from jax.experimental.pallas import tpu as pltpu
from jax.experimental.pallas import tpu_sc as plsc

_N = 10000
_E = 320000
_FIN = 128
_H = 112
_T = 16
_TN = 10
_C = 8

_NC = 2
_NS = 16
_NW = _NC * _NS
_EPW = _E // _NW
_ECH = 80
_NECH = _EPW // _ECH
_NBUF = 3
_RCH = 40
_NRCH = _N // _RCH
_DW = 16
_W = _H

_BLK = 1000
_NBLK = _N // _BLK

_SC_PARAMS = dict(compiler_params=pltpu.CompilerParams(use_tc_tiling_on_sc=False))


@functools.lru_cache(maxsize=None)
def _get_mesh():
    return plsc.VectorSubcoreMesh(core_axis_name="c", subcore_axis_name="s",
                                  num_cores=_NC, num_subcores=_NS)


def _zero_vmem_2d(ref, rows, cols):
    def body(i, _):
        for c in range(cols // 16):
            ref[i, pl.ds(c * 16, 16)] = jnp.zeros((16,), jnp.float32)
        return 0
    lax.fori_loop(0, rows, body, 0)


def _round_robin(sid, nch, fn):
    def body(k, _):
        c = sid + k * _NS
        @pl.when(c < nch)
        def _():
            fn(c)
        return 0
    lax.fori_loop(0, (nch + _NS - 1) // _NS, body, 0)


@functools.lru_cache(maxsize=None)
def _build_sc_degree():
    return functools.partial(
        pl.kernel,
        mesh=_get_mesh(),
        out_type=jax.ShapeDtypeStruct((_NC, _N, _DW), jnp.float32),
        scratch_types=[
            pltpu.VMEM((_NECH, _ECH), jnp.int32),
            pltpu.VMEM((_ECH, _DW), jnp.float32),
            pltpu.VMEM((_RCH, _DW), jnp.float32),
            pltpu.VMEM_SHARED((_N, _DW), jnp.float32),
        ],
        **_SC_PARAMS,
    )(_sc_degree_body)


def _sc_degree(dsts):
    return _build_sc_degree()(dsts)


def _sc_degree_body(dsts, out, idx_d, ones_v, zbuf, acc):
    cid = lax.axis_index("c")
    sid = lax.axis_index("s")
    def setup(i, _):
        for c in range(_DW // 16):
            ones_v[i, pl.ds(c * 16, 16)] = jnp.ones((16,), jnp.float32)
            zbuf[i, pl.ds(c * 16, 16)] = jnp.zeros((16,), jnp.float32)
        return 0
    lax.fori_loop(0, _ECH, setup, 0)

    _round_robin(sid, _NRCH,
                 lambda c: pltpu.sync_copy(zbuf, acc.at[pl.ds(c * _RCH, _RCH)]))
    pltpu.sync_copy(dsts.at[cid, sid], idx_d)
    plsc.subcore_barrier()

    def go(j, _):
        pltpu.sync_copy(ones_v, acc.at[idx_d.at[j]], add=True)
        return 0
    lax.fori_loop(0, _NECH, go, 0)
    plsc.subcore_barrier()

    _round_robin(sid, _NRCH,
                 lambda c: pltpu.sync_copy(acc.at[pl.ds(c * _RCH, _RCH)],
                                           out.at[cid, pl.ds(c * _RCH, _RCH)]))


@functools.lru_cache(maxsize=None)
def _build_sc_segsum():
    return functools.partial(
        pl.kernel,
        mesh=_get_mesh(),
        out_type=jax.ShapeDtypeStruct((_NC, _N, _W), jnp.float32),
        scratch_types=[
            pltpu.VMEM((_NECH, _ECH), jnp.int32),
            pltpu.VMEM((_NECH, _ECH), jnp.int32),
        ] + [pltpu.VMEM((_ECH, _W), jnp.float32) for _ in range(_NBUF)] + [
            pltpu.VMEM((_RCH, _W), jnp.float32),
            pltpu.VMEM_SHARED((_N, _W), jnp.float32),
        ] + [pltpu.SemaphoreType.DMA for _ in range(_NBUF)],
        **_SC_PARAMS,
    )(_sc_segsum_body)


def _sc_segsum(table, srcs, dsts):
    return _build_sc_segsum()(table, srcs, dsts)


def _sc_segsum_body(table, srcs, dsts, out, idx_s, idx_d, *rest):
    rows = rest[:_NBUF]
    zbuf, acc = rest[_NBUF], rest[_NBUF + 1]
    sems = rest[_NBUF + 2:]
    cid = lax.axis_index("c")
    sid = lax.axis_index("s")
    _zero_vmem_2d(zbuf, _RCH, _W)
    _round_robin(sid, _NRCH,
                 lambda c: pltpu.sync_copy(zbuf, acc.at[pl.ds(c * _RCH, _RCH)]))
    pltpu.sync_copy(srcs.at[cid, sid], idx_s)
    pltpu.sync_copy(dsts.at[cid, sid], idx_d)
    plsc.subcore_barrier()

    for b in range(_NBUF):
        pltpu.async_copy(table.at[idx_s.at[b]], rows[b], sems[b])

    def step(jj, b):
        pltpu.make_async_copy(table.at[idx_s.at[jj]], rows[b], sems[b]).wait()
        pltpu.sync_copy(rows[b], acc.at[idx_d.at[jj]], add=True)
        nxt = jj + _NBUF
        @pl.when(nxt < _NECH)
        def _():
            pltpu.async_copy(table.at[idx_s.at[nxt]], rows[b], sems[b])

    def go(jo, _):
        for b in range(_NBUF):
            step(jo * _NBUF + b, b)
        return 0
    _nfull = _NECH // _NBUF
    lax.fori_loop(0, _nfull, go, 0)
    for t in range(_NECH - _nfull * _NBUF):
        step(_nfull * _NBUF + t, t)
    plsc.subcore_barrier()

    _round_robin(sid, _NRCH,
                 lambda c: pltpu.sync_copy(acc.at[pl.ds(c * _RCH, _RCH)],
                                           out.at[cid, pl.ds(c * _RCH, _RCH)]))


_DOT = dict(preferred_element_type=jnp.float32, precision=lax.Precision.HIGHEST)


def _tc_pre_body(x_ref, w1_ref, degp_ref, u_ref, dinv_ref, indeg_ref,
                 sdeg_ref):
    pid = pl.program_id(0)
    indeg = degp_ref[0, :, 0:1] + degp_ref[1, :, 0:1]
    dinv = lax.rsqrt(indeg + 1.0)
    u_ref[...] = jnp.dot(x_ref[...], w1_ref[...], **_DOT) * dinv
    dinv_ref[...] = dinv
    indeg_ref[...] = indeg
    @pl.when(pid == 0)
    def _():
        sdeg_ref[...] = jnp.zeros_like(sdeg_ref)
    sdeg_ref[...] += jnp.sum(jnp.maximum(indeg, 1.0)).reshape(1, 1)


def _tc_relu_body(s1_ref, u_ref, dinv_ref, b1_ref, h_ref):
    s = s1_ref[0] + s1_ref[1] + u_ref[...]
    h_ref[...] = jnp.maximum(dinv_ref[...] * s + b1_ref[...], 0.0)


def _tc_de_body(h_ref, s2_ref, indeg_ref, sdeg_ref, tfeat_ref, tadj_ref,
                alpha0_ref, gamma_ref, beta_ref, w2_ref, dinv_ref,
                u2_ref, ysc, stat_sc):
    ph = pl.program_id(0)
    pid = pl.program_id(1)

    @pl.when(ph == 0)
    def _():
        indeg = indeg_ref[...]
        degc = jnp.maximum(indeg, 1.0)
        nb = (s2_ref[0] + s2_ref[1]) / degc
        tfeat = tfeat_ref[...]
        tmean = jnp.mean(tfeat, axis=1)
        msq = jnp.mean(jnp.sum(tfeat * tfeat, axis=2), axis=1)
        cross = lax.dot_general(nb, tmean, (((1,), (1,)), ((), ())), **_DOT)
        feat = (jnp.sum(nb * nb, axis=1, keepdims=True)
                - 2.0 * cross + msq[None, :])
        tstruct = jnp.mean(tadj_ref[...], axis=(1, 2))
        deg_norm = indeg * (_N / sdeg_ref[0, 0])
        struct = (deg_norm - tstruct[None, :]) ** 2
        alpha = jax.nn.sigmoid(alpha0_ref[0, 0])
        y = jnp.exp(-(alpha * feat + (1.0 - alpha) * struct))
        ysc[pl.ds(pid * _BLK, _BLK), :] = y
        h = h_ref[...]
        row0 = jnp.concatenate([jnp.sum(h, axis=0), jnp.sum(y, axis=0)])
        row1 = jnp.concatenate([jnp.sum(h * h, axis=0), jnp.sum(y * y, axis=0)])
        @pl.when(pid == 0)
        def _():
            stat_sc[...] = jnp.zeros_like(stat_sc)
        stat_sc[...] += jnp.stack([row0, row1])

    @pl.when(ph == 1)
    def _():
        mean = stat_sc[0:1, :] / _N
        var = stat_sc[1:2, :] / _N - mean * mean
        scale = lax.rsqrt(var + 1e-5) * gamma_ref[...]
        shift = beta_ref[...] - mean * scale
        z = jnp.concatenate([h_ref[...], ysc[pl.ds(pid * _BLK, _BLK), :]],
                            axis=1)
        zn = z * scale + shift
        p = jnp.dot(zn, w2_ref[...], **_DOT)
        u2_ref[...] = p * dinv_ref[...]


def _tc_final_body(s3_ref, u2_ref, dinv_ref, b2_ref, wl_ref, bl_ref,
                   out_ref, h2_ref):
    s = s3_ref[0] + s3_ref[1] + u2_ref[...]
    h2 = jnp.maximum(dinv_ref[...] * s + b2_ref[...], 0.0)
    h2_ref[...] = h2
    out_ref[...] = jnp.dot(h2, wl_ref[...], **_DOT) + bl_ref[...]


def _row_spec(cols):
    return pl.BlockSpec((_BLK, cols), lambda i: (i, 0))


def _part_spec(cols):
    return pl.BlockSpec((_NC, _BLK, cols), lambda i: (0, i, 0))


def _full_spec(shape):
    rank = len(shape)
    return pl.BlockSpec(shape, lambda i, _r=rank: (0,) * _r)


def kernel(x, edge_index, W1, b1, tfeat, tadj, alpha0, gamma, beta, W2, b2,
           Wlin, blin):
    f32 = jnp.float32
    src = edge_index[0].reshape(_NC, _NS, _NECH, _ECH).astype(jnp.int32)
    dst = edge_index[1].reshape(_NC, _NS, _NECH, _ECH).astype(jnp.int32)
    b1r = b1.reshape(1, _H)
    b2r = b2.reshape(1, _H)
    blr = blin.reshape(1, _C)
    gr = gamma.reshape(1, _H + _T)
    br = beta.reshape(1, _H + _T)
    a0 = alpha0.reshape(1, 1)

    degp = _sc_degree(dst)

    u, dinv, indeg, sdeg = pl.pallas_call(
        _tc_pre_body,
        grid=(_NBLK,),
        in_specs=[_row_spec(_FIN), _full_spec(W1.shape), _part_spec(_DW)],
        out_specs=[_row_spec(_H), _row_spec(1), _row_spec(1),
                   pl.BlockSpec((1, 1), lambda i: (0, 0))],
        out_shape=[jax.ShapeDtypeStruct((_N, _H), f32),
                   jax.ShapeDtypeStruct((_N, 1), f32),
                   jax.ShapeDtypeStruct((_N, 1), f32),
                   jax.ShapeDtypeStruct((1, 1), f32)],
    )(x, W1, degp)

    s1 = _sc_segsum(u, src, dst)

    h = pl.pallas_call(
        _tc_relu_body,
        grid=(_NBLK,),
        in_specs=[_part_spec(_H), _row_spec(_H), _row_spec(1),
                  _full_spec((1, _H))],
        out_specs=_row_spec(_H),
        out_shape=jax.ShapeDtypeStruct((_N, _H), f32),
    )(s1, u, dinv, b1r)

    s2 = _sc_segsum(h, src, dst)

    def _r2(cols):
        return pl.BlockSpec((_BLK, cols), lambda p, i: (i, 0))

    def _p2(cols):
        return pl.BlockSpec((_NC, _BLK, cols), lambda p, i: (0, i, 0))

    def _f2(shape):
        rank = len(shape)
        return pl.BlockSpec(shape, lambda p, i, _r=rank: (0,) * _r)

    u2 = pl.pallas_call(
        _tc_de_body,
        grid=(2, _NBLK),
        in_specs=[_r2(_H), _p2(_H), _r2(1), _f2((1, 1)), _f2(tfeat.shape),
                  _f2(tadj.shape), _f2((1, 1)), _f2((1, _H + _T)),
                  _f2((1, _H + _T)), _f2(W2.shape), _r2(1)],
        out_specs=_r2(_H),
        out_shape=jax.ShapeDtypeStruct((_N, _H), f32),
        scratch_shapes=[pltpu.VMEM((_N, _T), f32),
                        pltpu.VMEM((2, _H + _T), f32)],
    )(h, s2, indeg, sdeg, tfeat, tadj, a0, gr, br, W2, dinv)

    s3 = _sc_segsum(u2, src, dst)

    out, h2 = pl.pallas_call(
        _tc_final_body,
        grid=(_NBLK,),
        in_specs=[_part_spec(_H), _row_spec(_H), _row_spec(1),
                  _full_spec((1, _H)), _full_spec(Wlin.shape),
                  _full_spec((1, _C))],
        out_specs=[_row_spec(_C), _row_spec(_H)],
        out_shape=[jax.ShapeDtypeStruct((_N, _C), f32),
                   jax.ShapeDtypeStruct((_N, _H), f32)],
    )(s3, u2, dinv, b2r, Wlin, blr)

    return (out, h2)


# speedup vs baseline: 1.1772x; 1.0223x over previous
import functools
import jax
import jax.numpy as jnp
from jax import lax
from jax.experimental import pallas as pl
from jax.experimental.pallas import tpu as pltpu
from jax.experimental.pallas import tpu_sc as plsc

_N = 10000
_E = 320000
_FIN = 128
_H = 112
_T = 16
_TN = 10
_C = 8

_NC = 2
_NS = 16
_NW = _NC * _NS
_EPW = _E // _NW
_ECH = 80
_NECH = _EPW // _ECH
_NBUF = 4
_RCH = 40
_NRCH = _N // _RCH
_DW = 16
_W = _H

_BLK = 1000
_NBLK = _N // _BLK

_SC_PARAMS = dict(compiler_params=pltpu.CompilerParams(use_tc_tiling_on_sc=False))


@functools.lru_cache(maxsize=None)
def _get_mesh():
    return plsc.VectorSubcoreMesh(core_axis_name="c", subcore_axis_name="s",
                                  num_cores=_NC, num_subcores=_NS)


def _zero_vmem_2d(ref, rows, cols):
    def body(i, _):
        for c in range(cols // 16):
            ref[i, pl.ds(c * 16, 16)] = jnp.zeros((16,), jnp.float32)
        return 0
    lax.fori_loop(0, rows, body, 0)


def _round_robin(sid, nch, fn):
    def body(k, _):
        c = sid + k * _NS
        @pl.when(c < nch)
        def _():
            fn(c)
        return 0
    lax.fori_loop(0, (nch + _NS - 1) // _NS, body, 0)


@functools.lru_cache(maxsize=None)
def _build_sc_degree():
    return functools.partial(
        pl.kernel,
        mesh=_get_mesh(),
        out_type=jax.ShapeDtypeStruct((_NC, _N, _DW), jnp.float32),
        scratch_types=[
            pltpu.VMEM((_NECH, _ECH), jnp.int32),
            pltpu.VMEM((_ECH, _DW), jnp.float32),
            pltpu.VMEM((_RCH, _DW), jnp.float32),
            pltpu.VMEM_SHARED((_N, _DW), jnp.float32),
        ],
        **_SC_PARAMS,
    )(_sc_degree_body)


def _sc_degree(dsts):
    return _build_sc_degree()(dsts)


def _sc_degree_body(dsts, out, idx_d, ones_v, zbuf, acc):
    cid = lax.axis_index("c")
    sid = lax.axis_index("s")
    def setup(i, _):
        for c in range(_DW // 16):
            ones_v[i, pl.ds(c * 16, 16)] = jnp.ones((16,), jnp.float32)
            zbuf[i, pl.ds(c * 16, 16)] = jnp.zeros((16,), jnp.float32)
        return 0
    lax.fori_loop(0, _ECH, setup, 0)

    _round_robin(sid, _NRCH,
                 lambda c: pltpu.sync_copy(zbuf, acc.at[pl.ds(c * _RCH, _RCH)]))
    pltpu.sync_copy(dsts.at[cid, sid], idx_d)
    plsc.subcore_barrier()

    def go(j, _):
        pltpu.sync_copy(ones_v, acc.at[idx_d.at[j]], add=True)
        return 0
    lax.fori_loop(0, _NECH, go, 0)
    plsc.subcore_barrier()

    _round_robin(sid, _NRCH,
                 lambda c: pltpu.sync_copy(acc.at[pl.ds(c * _RCH, _RCH)],
                                           out.at[cid, pl.ds(c * _RCH, _RCH)]))


@functools.lru_cache(maxsize=None)
def _build_sc_segsum():
    return functools.partial(
        pl.kernel,
        mesh=_get_mesh(),
        out_type=jax.ShapeDtypeStruct((_NC, _N, _W), jnp.float32),
        scratch_types=[
            pltpu.VMEM((_NECH, _ECH), jnp.int32),
            pltpu.VMEM((_NECH, _ECH), jnp.int32),
        ] + [pltpu.VMEM((_ECH, _W), jnp.float32) for _ in range(_NBUF)] + [
            pltpu.VMEM((_RCH, _W), jnp.float32),
            pltpu.VMEM_SHARED((_N, _W), jnp.float32),
        ] + [pltpu.SemaphoreType.DMA for _ in range(_NBUF)],
        **_SC_PARAMS,
    )(_sc_segsum_body)


def _sc_segsum(table, srcs, dsts):
    return _build_sc_segsum()(table, srcs, dsts)


def _sc_segsum_body(table, srcs, dsts, out, idx_s, idx_d, *rest):
    rows = rest[:_NBUF]
    zbuf, acc = rest[_NBUF], rest[_NBUF + 1]
    sems = rest[_NBUF + 2:]
    cid = lax.axis_index("c")
    sid = lax.axis_index("s")
    _zero_vmem_2d(zbuf, _RCH, _W)
    _round_robin(sid, _NRCH,
                 lambda c: pltpu.sync_copy(zbuf, acc.at[pl.ds(c * _RCH, _RCH)]))
    pltpu.sync_copy(srcs.at[cid, sid], idx_s)
    pltpu.sync_copy(dsts.at[cid, sid], idx_d)
    plsc.subcore_barrier()

    for b in range(_NBUF):
        pltpu.async_copy(table.at[idx_s.at[b]], rows[b], sems[b])

    def step(jj, b):
        pltpu.make_async_copy(table.at[idx_s.at[jj]], rows[b], sems[b]).wait()
        pltpu.sync_copy(rows[b], acc.at[idx_d.at[jj]], add=True)
        nxt = jj + _NBUF
        @pl.when(nxt < _NECH)
        def _():
            pltpu.async_copy(table.at[idx_s.at[nxt]], rows[b], sems[b])

    def go(jo, _):
        for b in range(_NBUF):
            step(jo * _NBUF + b, b)
        return 0
    _nfull = _NECH // _NBUF
    lax.fori_loop(0, _nfull, go, 0)
    for t in range(_NECH - _nfull * _NBUF):
        step(_nfull * _NBUF + t, t)
    plsc.subcore_barrier()

    _round_robin(sid, _NRCH,
                 lambda c: pltpu.sync_copy(acc.at[pl.ds(c * _RCH, _RCH)],
                                           out.at[cid, pl.ds(c * _RCH, _RCH)]))


_DOT = dict(preferred_element_type=jnp.float32, precision=lax.Precision.HIGHEST)


def _tc_pre_body(x_ref, w1_ref, degp_ref, u_ref, dinv_ref, indeg_ref,
                 sdeg_ref):
    pid = pl.program_id(0)
    indeg = degp_ref[0, :, 0:1] + degp_ref[1, :, 0:1]
    dinv = lax.rsqrt(indeg + 1.0)
    u_ref[...] = jnp.dot(x_ref[...], w1_ref[...], **_DOT) * dinv
    dinv_ref[...] = dinv
    indeg_ref[...] = indeg
    @pl.when(pid == 0)
    def _():
        sdeg_ref[...] = jnp.zeros_like(sdeg_ref)
    sdeg_ref[...] += jnp.sum(jnp.maximum(indeg, 1.0)).reshape(1, 1)


def _tc_relu_body(s1_ref, u_ref, dinv_ref, b1_ref, h_ref):
    s = s1_ref[0] + s1_ref[1] + u_ref[...]
    h_ref[...] = jnp.maximum(dinv_ref[...] * s + b1_ref[...], 0.0)


def _tc_de_body(h_ref, s2_ref, indeg_ref, sdeg_ref, tfeat_ref, tadj_ref,
                alpha0_ref, gamma_ref, beta_ref, w2_ref, dinv_ref,
                u2_ref, ysc, stat_sc):
    ph = pl.program_id(0)
    pid = pl.program_id(1)

    @pl.when(ph == 0)
    def _():
        indeg = indeg_ref[...]
        degc = jnp.maximum(indeg, 1.0)
        nb = (s2_ref[0] + s2_ref[1]) / degc
        tfeat = tfeat_ref[...]
        tmean = jnp.mean(tfeat, axis=1)
        msq = jnp.mean(jnp.sum(tfeat * tfeat, axis=2), axis=1)
        cross = lax.dot_general(nb, tmean, (((1,), (1,)), ((), ())), **_DOT)
        feat = (jnp.sum(nb * nb, axis=1, keepdims=True)
                - 2.0 * cross + msq[None, :])
        tstruct = jnp.mean(tadj_ref[...], axis=(1, 2))
        deg_norm = indeg * (_N / sdeg_ref[0, 0])
        struct = (deg_norm - tstruct[None, :]) ** 2
        alpha = jax.nn.sigmoid(alpha0_ref[0, 0])
        y = jnp.exp(-(alpha * feat + (1.0 - alpha) * struct))
        ysc[pl.ds(pid * _BLK, _BLK), :] = y
        h = h_ref[...]
        row0 = jnp.concatenate([jnp.sum(h, axis=0), jnp.sum(y, axis=0)])
        row1 = jnp.concatenate([jnp.sum(h * h, axis=0), jnp.sum(y * y, axis=0)])
        @pl.when(pid == 0)
        def _():
            stat_sc[...] = jnp.zeros_like(stat_sc)
        stat_sc[...] += jnp.stack([row0, row1])

    @pl.when(ph == 1)
    def _():
        mean = stat_sc[0:1, :] / _N
        var = stat_sc[1:2, :] / _N - mean * mean
        scale = lax.rsqrt(var + 1e-5) * gamma_ref[...]
        shift = beta_ref[...] - mean * scale
        z = jnp.concatenate([h_ref[...], ysc[pl.ds(pid * _BLK, _BLK), :]],
                            axis=1)
        zn = z * scale + shift
        p = jnp.dot(zn, w2_ref[...], **_DOT)
        u2_ref[...] = p * dinv_ref[...]


def _tc_final_body(s3_ref, u2_ref, dinv_ref, b2_ref, wl_ref, bl_ref,
                   out_ref, h2_ref):
    s = s3_ref[0] + s3_ref[1] + u2_ref[...]
    h2 = jnp.maximum(dinv_ref[...] * s + b2_ref[...], 0.0)
    h2_ref[...] = h2
    out_ref[...] = jnp.dot(h2, wl_ref[...], **_DOT) + bl_ref[...]


def _row_spec(cols):
    return pl.BlockSpec((_BLK, cols), lambda i: (i, 0))


def _part_spec(cols):
    return pl.BlockSpec((_NC, _BLK, cols), lambda i: (0, i, 0))


def _full_spec(shape):
    rank = len(shape)
    return pl.BlockSpec(shape, lambda i, _r=rank: (0,) * _r)


def kernel(x, edge_index, W1, b1, tfeat, tadj, alpha0, gamma, beta, W2, b2,
           Wlin, blin):
    f32 = jnp.float32
    src = edge_index[0].reshape(_NC, _NS, _NECH, _ECH).astype(jnp.int32)
    dst = edge_index[1].reshape(_NC, _NS, _NECH, _ECH).astype(jnp.int32)
    b1r = b1.reshape(1, _H)
    b2r = b2.reshape(1, _H)
    blr = blin.reshape(1, _C)
    gr = gamma.reshape(1, _H + _T)
    br = beta.reshape(1, _H + _T)
    a0 = alpha0.reshape(1, 1)

    degp = _sc_degree(dst)

    u, dinv, indeg, sdeg = pl.pallas_call(
        _tc_pre_body,
        grid=(_NBLK,),
        in_specs=[_row_spec(_FIN), _full_spec(W1.shape), _part_spec(_DW)],
        out_specs=[_row_spec(_H), _row_spec(1), _row_spec(1),
                   pl.BlockSpec((1, 1), lambda i: (0, 0))],
        out_shape=[jax.ShapeDtypeStruct((_N, _H), f32),
                   jax.ShapeDtypeStruct((_N, 1), f32),
                   jax.ShapeDtypeStruct((_N, 1), f32),
                   jax.ShapeDtypeStruct((1, 1), f32)],
    )(x, W1, degp)

    s1 = _sc_segsum(u, src, dst)

    h = pl.pallas_call(
        _tc_relu_body,
        grid=(_NBLK,),
        in_specs=[_part_spec(_H), _row_spec(_H), _row_spec(1),
                  _full_spec((1, _H))],
        out_specs=_row_spec(_H),
        out_shape=jax.ShapeDtypeStruct((_N, _H), f32),
    )(s1, u, dinv, b1r)

    s2 = _sc_segsum(h, src, dst)

    def _r2(cols):
        return pl.BlockSpec((_BLK, cols), lambda p, i: (i, 0))

    def _p2(cols):
        return pl.BlockSpec((_NC, _BLK, cols), lambda p, i: (0, i, 0))

    def _f2(shape):
        rank = len(shape)
        return pl.BlockSpec(shape, lambda p, i, _r=rank: (0,) * _r)

    u2 = pl.pallas_call(
        _tc_de_body,
        grid=(2, _NBLK),
        in_specs=[_r2(_H), _p2(_H), _r2(1), _f2((1, 1)), _f2(tfeat.shape),
                  _f2(tadj.shape), _f2((1, 1)), _f2((1, _H + _T)),
                  _f2((1, _H + _T)), _f2(W2.shape), _r2(1)],
        out_specs=_r2(_H),
        out_shape=jax.ShapeDtypeStruct((_N, _H), f32),
        scratch_shapes=[pltpu.VMEM((_N, _T), f32),
                        pltpu.VMEM((2, _H + _T), f32)],
    )(h, s2, indeg, sdeg, tfeat, tadj, a0, gr, br, W2, dinv)

    s3 = _sc_segsum(u2, src, dst)

    out, h2 = pl.pallas_call(
        _tc_final_body,
        grid=(_NBLK,),
        in_specs=[_part_spec(_H), _row_spec(_H), _row_spec(1),
                  _full_spec((1, _H)), _full_spec(Wlin.shape),
                  _full_spec((1, _C))],
        out_specs=[_row_spec(_C), _row_spec(_H)],
        out_shape=[jax.ShapeDtypeStruct((_N, _C), f32),
                   jax.ShapeDtypeStruct((_N, _H), f32)],
    )(s3, u2, dinv, b2r, Wlin, blr)

    return (out, h2)


# speedup vs baseline: 1.2371x; 1.0508x over previous
import functools
import jax
import jax.numpy as jnp
from jax import lax
from jax.experimental import pallas as pl
from jax.experimental.pallas import tpu as pltpu
from jax.experimental.pallas import tpu_sc as plsc

_N = 10000
_E = 320000
_FIN = 128
_H = 112
_T = 16
_TN = 10
_C = 8

_NC = 2
_NS = 16
_NW = _NC * _NS
_EPW = _E // _NW
_ECH = 80
_NECH = _EPW // _ECH
_NBUF = 4
_RCH = 40
_NRCH = _N // _RCH
_DW = 16
_W = _H

_BLK = 1000
_NBLK = _N // _BLK

_SC_PARAMS = dict(compiler_params=pltpu.CompilerParams(use_tc_tiling_on_sc=False))


@functools.lru_cache(maxsize=None)
def _get_mesh():
    return plsc.VectorSubcoreMesh(core_axis_name="c", subcore_axis_name="s",
                                  num_cores=_NC, num_subcores=_NS)


def _zero_vmem_2d(ref, rows, cols):
    def body(i, _):
        for c in range(cols // 16):
            ref[i, pl.ds(c * 16, 16)] = jnp.zeros((16,), jnp.float32)
        return 0
    lax.fori_loop(0, rows, body, 0)


def _round_robin(sid, nch, fn):
    def body(k, _):
        c = sid + k * _NS
        @pl.when(c < nch)
        def _():
            fn(c)
        return 0
    lax.fori_loop(0, (nch + _NS - 1) // _NS, body, 0)


@functools.lru_cache(maxsize=None)
def _build_sc_degree():
    return functools.partial(
        pl.kernel,
        mesh=_get_mesh(),
        out_type=jax.ShapeDtypeStruct((_NC, _N, _DW), jnp.float32),
        scratch_types=[
            pltpu.VMEM((_NECH, _ECH), jnp.int32),
            pltpu.VMEM((_ECH, _DW), jnp.float32),
            pltpu.VMEM((_RCH, _DW), jnp.float32),
            pltpu.VMEM_SHARED((_N, _DW), jnp.float32),
        ],
        **_SC_PARAMS,
    )(_sc_degree_body)


def _sc_degree(dsts):
    return _build_sc_degree()(dsts)


def _sc_degree_body(dsts, out, idx_d, ones_v, zbuf, acc):
    cid = lax.axis_index("c")
    sid = lax.axis_index("s")
    def setup(i, _):
        for c in range(_DW // 16):
            ones_v[i, pl.ds(c * 16, 16)] = jnp.ones((16,), jnp.float32)
            zbuf[i, pl.ds(c * 16, 16)] = jnp.zeros((16,), jnp.float32)
        return 0
    lax.fori_loop(0, _ECH, setup, 0)

    _round_robin(sid, _NRCH,
                 lambda c: pltpu.sync_copy(zbuf, acc.at[pl.ds(c * _RCH, _RCH)]))
    pltpu.sync_copy(dsts.at[cid, sid], idx_d)
    plsc.subcore_barrier()

    def go(j, _):
        pltpu.sync_copy(ones_v, acc.at[idx_d.at[j]], add=True)
        return 0
    lax.fori_loop(0, _NECH, go, 0)
    plsc.subcore_barrier()

    _round_robin(sid, _NRCH,
                 lambda c: pltpu.sync_copy(acc.at[pl.ds(c * _RCH, _RCH)],
                                           out.at[cid, pl.ds(c * _RCH, _RCH)]))


@functools.lru_cache(maxsize=None)
def _build_sc_segsum():
    return functools.partial(
        pl.kernel,
        mesh=_get_mesh(),
        out_type=jax.ShapeDtypeStruct((_NC, _N, _W), jnp.float32),
        scratch_types=[
            pltpu.VMEM((_NECH, _ECH), jnp.int32),
            pltpu.VMEM((_NECH, _ECH), jnp.int32),
        ] + [pltpu.VMEM((_ECH, _W), jnp.float32) for _ in range(_NBUF)] + [
            pltpu.VMEM((_RCH, _W), jnp.float32),
            pltpu.VMEM_SHARED((_N, _W), jnp.float32),
        ] + [pltpu.SemaphoreType.DMA for _ in range(_NBUF + 1)],
        **_SC_PARAMS,
    )(_sc_segsum_body)


def _sc_segsum(table, srcs, dsts):
    return _build_sc_segsum()(table, srcs, dsts)


def _sc_segsum_body(table, srcs, dsts, out, idx_s, idx_d, *rest):
    rows = rest[:_NBUF]
    zbuf, acc = rest[_NBUF], rest[_NBUF + 1]
    sems = rest[_NBUF + 2:_NBUF + 2 + _NBUF]
    csem = rest[_NBUF + 2 + _NBUF]
    cid = lax.axis_index("c")
    sid = lax.axis_index("s")
    _zero_vmem_2d(zbuf, _RCH, _W)
    _round_robin(sid, _NRCH,
                 lambda c: pltpu.async_copy(zbuf, acc.at[pl.ds(c * _RCH, _RCH)],
                                            csem))
    pltpu.sync_copy(srcs.at[cid, sid], idx_s)
    pltpu.sync_copy(dsts.at[cid, sid], idx_d)
    _round_robin(sid, _NRCH,
                 lambda c: pltpu.make_async_copy(
                     zbuf, acc.at[pl.ds(c * _RCH, _RCH)], csem).wait())
    plsc.subcore_barrier()

    for b in range(_NBUF):
        pltpu.async_copy(table.at[idx_s.at[b]], rows[b], sems[b])

    def step(jj, b):
        pltpu.make_async_copy(table.at[idx_s.at[jj]], rows[b], sems[b]).wait()
        pltpu.sync_copy(rows[b], acc.at[idx_d.at[jj]], add=True)
        nxt = jj + _NBUF
        @pl.when(nxt < _NECH)
        def _():
            pltpu.async_copy(table.at[idx_s.at[nxt]], rows[b], sems[b])

    def go(jo, _):
        for b in range(_NBUF):
            step(jo * _NBUF + b, b)
        return 0
    _nfull = _NECH // _NBUF
    lax.fori_loop(0, _nfull, go, 0)
    for t in range(_NECH - _nfull * _NBUF):
        step(_nfull * _NBUF + t, t)
    plsc.subcore_barrier()

    _round_robin(sid, _NRCH,
                 lambda c: pltpu.async_copy(
                     acc.at[pl.ds(c * _RCH, _RCH)],
                     out.at[cid, pl.ds(c * _RCH, _RCH)], csem))
    _round_robin(sid, _NRCH,
                 lambda c: pltpu.make_async_copy(
                     acc.at[pl.ds(c * _RCH, _RCH)],
                     out.at[cid, pl.ds(c * _RCH, _RCH)], csem).wait())


_DOT = dict(preferred_element_type=jnp.float32, precision=lax.Precision.HIGHEST)


def _tc_pre_body(x_ref, w1_ref, degp_ref, u_ref, dinv_ref, indeg_ref,
                 sdeg_ref):
    pid = pl.program_id(0)
    indeg = degp_ref[0, :, 0:1] + degp_ref[1, :, 0:1]
    dinv = lax.rsqrt(indeg + 1.0)
    u_ref[...] = jnp.dot(x_ref[...], w1_ref[...], **_DOT) * dinv
    dinv_ref[...] = dinv
    indeg_ref[...] = indeg
    @pl.when(pid == 0)
    def _():
        sdeg_ref[...] = jnp.zeros_like(sdeg_ref)
    sdeg_ref[...] += jnp.sum(jnp.maximum(indeg, 1.0)).reshape(1, 1)


def _tc_relu_body(s1_ref, u_ref, dinv_ref, b1_ref, h_ref):
    s = s1_ref[0] + s1_ref[1] + u_ref[...]
    h_ref[...] = jnp.maximum(dinv_ref[...] * s + b1_ref[...], 0.0)


def _tc_de_body(h_ref, s2_ref, indeg_ref, sdeg_ref, tfeat_ref, tadj_ref,
                alpha0_ref, gamma_ref, beta_ref, w2_ref, dinv_ref,
                u2_ref, ysc, stat_sc):
    ph = pl.program_id(0)
    pid = pl.program_id(1)

    @pl.when(ph == 0)
    def _():
        indeg = indeg_ref[...]
        degc = jnp.maximum(indeg, 1.0)
        nb = (s2_ref[0] + s2_ref[1]) / degc
        tfeat = tfeat_ref[...]
        tmean = jnp.mean(tfeat, axis=1)
        msq = jnp.mean(jnp.sum(tfeat * tfeat, axis=2), axis=1)
        cross = lax.dot_general(nb, tmean, (((1,), (1,)), ((), ())), **_DOT)
        feat = (jnp.sum(nb * nb, axis=1, keepdims=True)
                - 2.0 * cross + msq[None, :])
        tstruct = jnp.mean(tadj_ref[...], axis=(1, 2))
        deg_norm = indeg * (_N / sdeg_ref[0, 0])
        struct = (deg_norm - tstruct[None, :]) ** 2
        alpha = jax.nn.sigmoid(alpha0_ref[0, 0])
        y = jnp.exp(-(alpha * feat + (1.0 - alpha) * struct))
        ysc[pl.ds(pid * _BLK, _BLK), :] = y
        h = h_ref[...]
        row0 = jnp.concatenate([jnp.sum(h, axis=0), jnp.sum(y, axis=0)])
        row1 = jnp.concatenate([jnp.sum(h * h, axis=0), jnp.sum(y * y, axis=0)])
        @pl.when(pid == 0)
        def _():
            stat_sc[...] = jnp.zeros_like(stat_sc)
        stat_sc[...] += jnp.stack([row0, row1])

    @pl.when(ph == 1)
    def _():
        mean = stat_sc[0:1, :] / _N
        var = stat_sc[1:2, :] / _N - mean * mean
        scale = lax.rsqrt(var + 1e-5) * gamma_ref[...]
        shift = beta_ref[...] - mean * scale
        z = jnp.concatenate([h_ref[...], ysc[pl.ds(pid * _BLK, _BLK), :]],
                            axis=1)
        zn = z * scale + shift
        p = jnp.dot(zn, w2_ref[...], **_DOT)
        u2_ref[...] = p * dinv_ref[...]


def _tc_final_body(s3_ref, u2_ref, dinv_ref, b2_ref, wl_ref, bl_ref,
                   out_ref, h2_ref):
    s = s3_ref[0] + s3_ref[1] + u2_ref[...]
    h2 = jnp.maximum(dinv_ref[...] * s + b2_ref[...], 0.0)
    h2_ref[...] = h2
    out_ref[...] = jnp.dot(h2, wl_ref[...], **_DOT) + bl_ref[...]


def _row_spec(cols):
    return pl.BlockSpec((_BLK, cols), lambda i: (i, 0))


def _part_spec(cols):
    return pl.BlockSpec((_NC, _BLK, cols), lambda i: (0, i, 0))


def _full_spec(shape):
    rank = len(shape)
    return pl.BlockSpec(shape, lambda i, _r=rank: (0,) * _r)


def kernel(x, edge_index, W1, b1, tfeat, tadj, alpha0, gamma, beta, W2, b2,
           Wlin, blin):
    f32 = jnp.float32
    src = edge_index[0].reshape(_NC, _NS, _NECH, _ECH).astype(jnp.int32)
    dst = edge_index[1].reshape(_NC, _NS, _NECH, _ECH).astype(jnp.int32)
    b1r = b1.reshape(1, _H)
    b2r = b2.reshape(1, _H)
    blr = blin.reshape(1, _C)
    gr = gamma.reshape(1, _H + _T)
    br = beta.reshape(1, _H + _T)
    a0 = alpha0.reshape(1, 1)

    degp = _sc_degree(dst)

    u, dinv, indeg, sdeg = pl.pallas_call(
        _tc_pre_body,
        grid=(_NBLK,),
        in_specs=[_row_spec(_FIN), _full_spec(W1.shape), _part_spec(_DW)],
        out_specs=[_row_spec(_H), _row_spec(1), _row_spec(1),
                   pl.BlockSpec((1, 1), lambda i: (0, 0))],
        out_shape=[jax.ShapeDtypeStruct((_N, _H), f32),
                   jax.ShapeDtypeStruct((_N, 1), f32),
                   jax.ShapeDtypeStruct((_N, 1), f32),
                   jax.ShapeDtypeStruct((1, 1), f32)],
    )(x, W1, degp)

    s1 = _sc_segsum(u, src, dst)

    h = pl.pallas_call(
        _tc_relu_body,
        grid=(_NBLK,),
        in_specs=[_part_spec(_H), _row_spec(_H), _row_spec(1),
                  _full_spec((1, _H))],
        out_specs=_row_spec(_H),
        out_shape=jax.ShapeDtypeStruct((_N, _H), f32),
    )(s1, u, dinv, b1r)

    s2 = _sc_segsum(h, src, dst)

    def _r2(cols):
        return pl.BlockSpec((_BLK, cols), lambda p, i: (i, 0))

    def _p2(cols):
        return pl.BlockSpec((_NC, _BLK, cols), lambda p, i: (0, i, 0))

    def _f2(shape):
        rank = len(shape)
        return pl.BlockSpec(shape, lambda p, i, _r=rank: (0,) * _r)

    u2 = pl.pallas_call(
        _tc_de_body,
        grid=(2, _NBLK),
        in_specs=[_r2(_H), _p2(_H), _r2(1), _f2((1, 1)), _f2(tfeat.shape),
                  _f2(tadj.shape), _f2((1, 1)), _f2((1, _H + _T)),
                  _f2((1, _H + _T)), _f2(W2.shape), _r2(1)],
        out_specs=_r2(_H),
        out_shape=jax.ShapeDtypeStruct((_N, _H), f32),
        scratch_shapes=[pltpu.VMEM((_N, _T), f32),
                        pltpu.VMEM((2, _H + _T), f32)],
    )(h, s2, indeg, sdeg, tfeat, tadj, a0, gr, br, W2, dinv)

    s3 = _sc_segsum(u2, src, dst)

    out, h2 = pl.pallas_call(
        _tc_final_body,
        grid=(_NBLK,),
        in_specs=[_part_spec(_H), _row_spec(_H), _row_spec(1),
                  _full_spec((1, _H)), _full_spec(Wlin.shape),
                  _full_spec((1, _C))],
        out_specs=[_row_spec(_C), _row_spec(_H)],
        out_shape=[jax.ShapeDtypeStruct((_N, _C), f32),
                   jax.ShapeDtypeStruct((_N, _H), f32)],
    )(s3, u2, dinv, b2r, Wlin, blr)

    return (out, h2)
